# Initial kernel scaffold; baseline (speedup 1.0000x reference)
import jax
import jax.numpy as jnp
from jax import lax
from jax.experimental import pallas as pl
from jax.experimental.pallas import tpu as pltpu
from jax.experimental.pallas import tpu_sc as plsc

N = 10000
PAD = 16
P = N + PAD
D = 16
WIN = 128
NC = 2
NS = 16
NW = NC * NS
ROWS_PER_TILE = P // NS


def _ceil_to(x, m):
    return (x + m - 1) // m * m


def _make_sc_pass(wpw, with_counts):
    mesh = plsc.VectorSubcoreMesh(core_axis_name="c", subcore_axis_name="s")
    n_out = 3 if with_counts else 1
    out_type = [jax.ShapeDtypeStruct((NC, P, D), jnp.float32)] * n_out
    scratch = [
        pltpu.VMEM_SHARED((P, D), jnp.float32),
        pltpu.VMEM((wpw, WIN), jnp.int32),
        pltpu.VMEM((wpw, WIN), jnp.int32),
        pltpu.VMEM((WIN, D), jnp.float32),
        pltpu.VMEM((ROWS_PER_TILE, D), jnp.float32),
    ]
    if with_counts:
        scratch += [
            pltpu.VMEM_SHARED((P, D), jnp.float32),
            pltpu.VMEM_SHARED((P, D), jnp.float32),
            pltpu.VMEM((WIN, D), jnp.float32),
        ]

    def body(table_hbm, gidx_hbm, sidx_hbm, *refs):
        outs = refs[:n_out]
        acc_sp, gidx_v, sidx_v, rows_v, slab_v = refs[n_out:n_out + 5]
        if with_counts:
            de_sp, dv_sp, ones_v = refs[n_out + 5:]
        cid = lax.axis_index("c")
        sid = lax.axis_index("s")
        wid = cid * NS + sid
        row0 = sid * ROWS_PER_TILE

        def zrow(i, _):
            slab_v[i, :] = jnp.zeros((D,), jnp.float32)
            return 0
        lax.fori_loop(0, ROWS_PER_TILE, zrow, 0)
        pltpu.sync_copy(slab_v, acc_sp.at[pl.ds(row0, ROWS_PER_TILE)])
        if with_counts:
            pltpu.sync_copy(slab_v, de_sp.at[pl.ds(row0, ROWS_PER_TILE)])
            pltpu.sync_copy(slab_v, dv_sp.at[pl.ds(row0, ROWS_PER_TILE)])

            def orow(i, _):
                ones_v[i, :] = jnp.ones((D,), jnp.float32)
                return 0
            lax.fori_loop(0, WIN, orow, 0)

        base = wid * wpw
        pltpu.sync_copy(gidx_hbm.at[pl.ds(base, wpw)], gidx_v)
        pltpu.sync_copy(sidx_hbm.at[pl.ds(base, wpw)], sidx_v)

        plsc.subcore_barrier()

        def window(j, _):
            pltpu.sync_copy(table_hbm.at[gidx_v.at[j]], rows_v)
            pltpu.sync_copy(rows_v, acc_sp.at[sidx_v.at[j]], add=True)
            if with_counts:
                pltpu.sync_copy(ones_v, de_sp.at[sidx_v.at[j]], add=True)
                pltpu.sync_copy(ones_v, dv_sp.at[gidx_v.at[j]], add=True)
            return 0
        lax.fori_loop(0, wpw, window, 0)

        plsc.subcore_barrier()

        srcs = (acc_sp, de_sp, dv_sp) if with_counts else (acc_sp,)
        for out_hbm, src in zip(outs, srcs):
            pltpu.sync_copy(src.at[pl.ds(row0, ROWS_PER_TILE)], slab_v)
            pltpu.sync_copy(slab_v, out_hbm.at[cid, pl.ds(row0, ROWS_PER_TILE)])

    return pl.kernel(body, out_type=out_type, mesh=mesh, scratch_types=scratch)


def _tc_call(f, out_shapes):
    return pl.pallas_call(f, out_shape=out_shapes)


def _proj_body(x_ref, w_ref, b_ref, o_ref):
    o_ref[...] = (
        jnp.dot(x_ref[...], w_ref[...], preferred_element_type=jnp.float32)
        + b_ref[...]
    )


def _combine1_body(acc_ref, de_ref, dv_ref, ef_ref, rde_ref, rdv_ref, m_ref):
    de = de_ref[0] + de_ref[1]
    dv = dv_ref[0] + dv_ref[1]
    rde = 1.0 / jnp.maximum(de, 1.0)
    rdv = 1.0 / jnp.maximum(dv, 1.0)
    ef_ref[...] = (acc_ref[0] + acc_ref[1]) * rde
    rde_ref[...] = rde
    rdv_ref[...] = rdv
    m_ref[...] = (dv > 0.0).astype(jnp.float32)


def _scale_leaky_body(acc_ref, rdv_ref, o_ref):
    t = (acc_ref[0] + acc_ref[1]) * rdv_ref[...]
    o_ref[...] = jnp.maximum(t, 0.01 * t)


def _scale_body(acc_ref, rde_ref, o_ref):
    o_ref[...] = (acc_ref[0] + acc_ref[1]) * rde_ref[...]


def _final_body(acc_ref, rdv_ref, m_ref, w2_ref, b2_ref, o_ref):
    n2 = (acc_ref[0] + acc_ref[1]) * rdv_ref[...]
    logits = jnp.dot(
        n2[:N], w2_ref[...], preferred_element_type=jnp.float32
    ) + m_ref[:N, 0:1] * b2_ref[...]
    z = logits - jnp.max(logits, axis=1, keepdims=True)
    o_ref[...] = z - jnp.log(jnp.sum(jnp.exp(z), axis=1, keepdims=True))


def kernel(x, H, W1, b1, W2, b2):
    n_class = W2.shape[1]
    ni = H.shape[1]
    nip = _ceil_to(ni, NW * WIN)
    wpw = nip // (NW * WIN)

    pad = nip - ni
    pad_idx = (N + (jnp.arange(pad, dtype=jnp.int32) % PAD))[None, :]
    Hp = jnp.concatenate([H.astype(jnp.int32), jnp.tile(pad_idx, (2, 1))], axis=1)
    nidx = Hp[0].reshape(nip // WIN, WIN)
    eidx = Hp[1].reshape(nip // WIN, WIN)

    xpad = jnp.pad(x, ((0, P - N), (0, 0)))

    xp = _tc_call(_proj_body, jax.ShapeDtypeStruct((P, D), jnp.float32))(
        xpad, W1, b1.reshape(1, D)
    )

    sc_counts = _make_sc_pass(wpw, with_counts=True)
    sc_plain = _make_sc_pass(wpw, with_counts=False)

    accA, deP, dvP = sc_counts(xp, nidx, eidx)

    ef1, rde, rdv, mask = _tc_call(
        _combine1_body,
        [jax.ShapeDtypeStruct((P, D), jnp.float32)] * 4,
    )(accA, deP, dvP)

    accB = sc_plain(ef1, eidx, nidx)
    if isinstance(accB, (list, tuple)):
        accB = accB[0]

    h = _tc_call(_scale_leaky_body, jax.ShapeDtypeStruct((P, D), jnp.float32))(
        accB, rdv
    )

    accC = sc_plain(h, nidx, eidx)
    if isinstance(accC, (list, tuple)):
        accC = accC[0]

    ef2 = _tc_call(_scale_body, jax.ShapeDtypeStruct((P, D), jnp.float32))(
        accC, rde
    )

    accD = sc_plain(ef2, eidx, nidx)
    if isinstance(accD, (list, tuple)):
        accD = accD[0]

    out = _tc_call(_final_body, jax.ShapeDtypeStruct((N, n_class), jnp.float32))(
        accD, rdv, mask, W2, b2.reshape(1, n_class)
    )
    return out


# speedup vs baseline: 11.9566x; 11.9566x over previous
import jax
import jax.numpy as jnp
from jax import lax
from jax.experimental import pallas as pl
from jax.experimental.pallas import tpu as pltpu
from jax.experimental.pallas import tpu_sc as plsc

N = 10000
PAD = 240
P = N + PAD
D = 16
WIN = 128
NC = 2
NS = 16
NW = NC * NS
ROWS_PER_TILE = P // NS


def _ceil_to(x, m):
    return (x + m - 1) // m * m


def _make_sc_pass(wpw, with_counts):
    mesh = plsc.VectorSubcoreMesh(
        core_axis_name="c", subcore_axis_name="s", num_cores=NC, num_subcores=NS
    )
    n_out = 3 if with_counts else 1
    out_type = [jax.ShapeDtypeStruct((NC, P, D), jnp.float32)] * n_out
    scratch = [
        pltpu.VMEM_SHARED((P, D), jnp.float32),
        pltpu.VMEM((wpw, WIN), jnp.int32),
        pltpu.VMEM((wpw, WIN), jnp.int32),
        pltpu.VMEM((WIN, D), jnp.float32),
        pltpu.VMEM((ROWS_PER_TILE, D), jnp.float32),
    ]
    if with_counts:
        scratch += [
            pltpu.VMEM_SHARED((P, D), jnp.float32),
            pltpu.VMEM_SHARED((P, D), jnp.float32),
            pltpu.VMEM((WIN, D), jnp.float32),
        ]

    def body(table_hbm, gidx_hbm, sidx_hbm, *refs):
        outs = refs[:n_out]
        acc_sp, gidx_v, sidx_v, rows_v, slab_v = refs[n_out:n_out + 5]
        if with_counts:
            de_sp, dv_sp, ones_v = refs[n_out + 5:]
        cid = lax.axis_index("c")
        sid = lax.axis_index("s")
        wid = cid * NS + sid
        row0 = sid * ROWS_PER_TILE

        def zrow(i, _):
            slab_v[i, :] = jnp.zeros((D,), jnp.float32)
            return 0
        lax.fori_loop(0, ROWS_PER_TILE, zrow, 0)
        pltpu.sync_copy(slab_v, acc_sp.at[pl.ds(row0, ROWS_PER_TILE)])
        if with_counts:
            pltpu.sync_copy(slab_v, de_sp.at[pl.ds(row0, ROWS_PER_TILE)])
            pltpu.sync_copy(slab_v, dv_sp.at[pl.ds(row0, ROWS_PER_TILE)])

            def orow(i, _):
                ones_v[i, :] = jnp.ones((D,), jnp.float32)
                return 0
            lax.fori_loop(0, WIN, orow, 0)

        base = wid * wpw
        pltpu.sync_copy(gidx_hbm.at[pl.ds(base, wpw)], gidx_v)
        pltpu.sync_copy(sidx_hbm.at[pl.ds(base, wpw)], sidx_v)

        plsc.subcore_barrier()

        def window(j, _):
            pltpu.sync_copy(table_hbm.at[gidx_v.at[j]], rows_v)
            pltpu.sync_copy(rows_v, acc_sp.at[sidx_v.at[j]], add=True)
            if with_counts:
                pltpu.sync_copy(ones_v, de_sp.at[sidx_v.at[j]], add=True)
                pltpu.sync_copy(ones_v, dv_sp.at[gidx_v.at[j]], add=True)
            return 0
        lax.fori_loop(0, wpw, window, 0)

        plsc.subcore_barrier()

        srcs = (acc_sp, de_sp, dv_sp) if with_counts else (acc_sp,)
        for out_hbm, src in zip(outs, srcs):
            pltpu.sync_copy(src.at[pl.ds(row0, ROWS_PER_TILE)], slab_v)
            pltpu.sync_copy(slab_v, out_hbm.at[cid, pl.ds(row0, ROWS_PER_TILE)])

    return pl.kernel(
        body,
        out_type=out_type,
        mesh=mesh,
        scratch_types=scratch,
        compiler_params=pltpu.CompilerParams(use_tc_tiling_on_sc=False),
    )


def _tc_call(f, out_shapes):
    return pl.pallas_call(f, out_shape=out_shapes)


def _proj_body(x_ref, w_ref, b_ref, o_ref):
    o_ref[...] = (
        jnp.dot(x_ref[...], w_ref[...], preferred_element_type=jnp.float32)
        + b_ref[...]
    )


def _combine1_body(acc_ref, de_ref, dv_ref, ef_ref, rde_ref, rdv_ref, m_ref):
    de = de_ref[0] + de_ref[1]
    dv = dv_ref[0] + dv_ref[1]
    rde = 1.0 / jnp.maximum(de, 1.0)
    rdv = 1.0 / jnp.maximum(dv, 1.0)
    ef_ref[...] = (acc_ref[0] + acc_ref[1]) * rde
    rde_ref[...] = rde
    rdv_ref[...] = rdv
    m_ref[...] = (dv > 0.0).astype(jnp.float32)


def _scale_leaky_body(acc_ref, rdv_ref, o_ref):
    t = (acc_ref[0] + acc_ref[1]) * rdv_ref[...]
    o_ref[...] = jnp.maximum(t, 0.01 * t)


def _scale_body(acc_ref, rde_ref, o_ref):
    o_ref[...] = (acc_ref[0] + acc_ref[1]) * rde_ref[...]


def _final_body(acc_ref, rdv_ref, m_ref, w2_ref, b2_ref, o_ref):
    n2 = (acc_ref[0] + acc_ref[1]) * rdv_ref[...]
    logits = jnp.dot(
        n2[:N], w2_ref[...], preferred_element_type=jnp.float32
    ) + m_ref[:N, 0:1] * b2_ref[...]
    z = logits - jnp.max(logits, axis=1, keepdims=True)
    o_ref[...] = z - jnp.log(jnp.sum(jnp.exp(z), axis=1, keepdims=True))


def kernel(x, H, W1, b1, W2, b2):
    n_class = W2.shape[1]
    ni = H.shape[1]
    nip = _ceil_to(ni, NW * WIN * 8)
    wpw = nip // (NW * WIN)

    pad = nip - ni
    pad_idx = (N + (jnp.arange(pad, dtype=jnp.int32) % PAD))[None, :]
    Hp = jnp.concatenate([H.astype(jnp.int32), jnp.tile(pad_idx, (2, 1))], axis=1)
    nidx = Hp[0].reshape(nip // WIN, WIN)
    eidx = Hp[1].reshape(nip // WIN, WIN)

    xpad = jnp.pad(x, ((0, P - N), (0, 0)))

    xp = _tc_call(_proj_body, jax.ShapeDtypeStruct((P, D), jnp.float32))(
        xpad, W1, b1.reshape(1, D)
    )

    sc_counts = _make_sc_pass(wpw, with_counts=True)
    sc_plain = _make_sc_pass(wpw, with_counts=False)

    accA, deP, dvP = sc_counts(xp, nidx, eidx)

    ef1, rde, rdv, mask = _tc_call(
        _combine1_body,
        [jax.ShapeDtypeStruct((P, D), jnp.float32)] * 4,
    )(accA, deP, dvP)

    accB = sc_plain(ef1, eidx, nidx)
    if isinstance(accB, (list, tuple)):
        accB = accB[0]

    h = _tc_call(_scale_leaky_body, jax.ShapeDtypeStruct((P, D), jnp.float32))(
        accB, rdv
    )

    accC = sc_plain(h, nidx, eidx)
    if isinstance(accC, (list, tuple)):
        accC = accC[0]

    ef2 = _tc_call(_scale_body, jax.ShapeDtypeStruct((P, D), jnp.float32))(
        accC, rde
    )

    accD = sc_plain(ef2, eidx, nidx)
    if isinstance(accD, (list, tuple)):
        accD = accD[0]

    out = _tc_call(_final_body, jax.ShapeDtypeStruct((N, n_class), jnp.float32))(
        accD, rdv, mask, W2, b2.reshape(1, n_class)
    )
    return out


# speedup vs baseline: 16.9553x; 1.4181x over previous
import jax
import jax.numpy as jnp
from jax import lax
from jax.experimental import pallas as pl
from jax.experimental.pallas import tpu as pltpu
from jax.experimental.pallas import tpu_sc as plsc

N = 10000
PAD = 240
P = N + PAD
D = 16
WIN = 128
NC = 2
NS = 16
NW = NC * NS
ROWS_PER_TILE = P // NS


def _ceil_to(x, m):
    return (x + m - 1) // m * m


def _make_sc_pass(wpw, with_counts):
    mesh = plsc.VectorSubcoreMesh(
        core_axis_name="c", subcore_axis_name="s", num_cores=NC, num_subcores=NS
    )
    n_out = 3 if with_counts else 1
    out_type = [jax.ShapeDtypeStruct((NC, P, D), jnp.float32)] * n_out
    scratch = [
        pltpu.VMEM_SHARED((P, D), jnp.float32),
        pltpu.VMEM((wpw, WIN), jnp.int32),
        pltpu.VMEM((wpw, WIN), jnp.int32),
        pltpu.VMEM((WIN, D), jnp.float32),
        pltpu.VMEM((WIN, D), jnp.float32),
        pltpu.VMEM((ROWS_PER_TILE, D), jnp.float32),
        pltpu.SemaphoreType.DMA,
        pltpu.SemaphoreType.DMA,
    ]
    if with_counts:
        scratch += [
            pltpu.VMEM_SHARED((P, D), jnp.float32),
            pltpu.VMEM_SHARED((P, D), jnp.float32),
            pltpu.VMEM((WIN, D), jnp.float32),
        ]

    def body(table_hbm, gidx_hbm, sidx_hbm, *refs):
        outs = refs[:n_out]
        (acc_sp, gidx_v, sidx_v, rows0_v, rows1_v, slab_v, sem0, sem1) = refs[
            n_out:n_out + 8
        ]
        if with_counts:
            de_sp, dv_sp, ones_v = refs[n_out + 8:]
        cid = lax.axis_index("c")
        sid = lax.axis_index("s")
        wid = cid * NS + sid
        row0 = sid * ROWS_PER_TILE

        def zrow(i, _):
            slab_v[i, :] = jnp.zeros((D,), jnp.float32)
            return 0
        lax.fori_loop(0, ROWS_PER_TILE, zrow, 0)
        pltpu.sync_copy(slab_v, acc_sp.at[pl.ds(row0, ROWS_PER_TILE)])
        if with_counts:
            pltpu.sync_copy(slab_v, de_sp.at[pl.ds(row0, ROWS_PER_TILE)])
            pltpu.sync_copy(slab_v, dv_sp.at[pl.ds(row0, ROWS_PER_TILE)])

            def orow(i, _):
                ones_v[i, :] = jnp.ones((D,), jnp.float32)
                return 0
            lax.fori_loop(0, WIN, orow, 0)

        base = wid * wpw
        pltpu.sync_copy(gidx_hbm.at[pl.ds(base, wpw)], gidx_v)
        pltpu.sync_copy(sidx_hbm.at[pl.ds(base, wpw)], sidx_v)

        plsc.subcore_barrier()

        def gather(j, buf, sem):
            return pltpu.make_async_copy(table_hbm.at[gidx_v.at[j]], buf, sem)

        def scatter(j, buf):
            pltpu.sync_copy(buf, acc_sp.at[sidx_v.at[j]], add=True)
            if with_counts:
                pltpu.sync_copy(ones_v, de_sp.at[sidx_v.at[j]], add=True)
                pltpu.sync_copy(ones_v, dv_sp.at[gidx_v.at[j]], add=True)

        gather(0, rows0_v, sem0).start()

        def window2(k, _):
            a = 2 * k
            b = a + 1
            gather(b, rows1_v, sem1).start()
            gather(a, rows0_v, sem0).wait()
            scatter(a, rows0_v)

            @pl.when(a + 2 < wpw)
            def _():
                gather(a + 2, rows0_v, sem0).start()

            gather(b, rows1_v, sem1).wait()
            scatter(b, rows1_v)
            return 0
        lax.fori_loop(0, wpw // 2, window2, 0)

        plsc.subcore_barrier()

        srcs = (acc_sp, de_sp, dv_sp) if with_counts else (acc_sp,)
        for out_hbm, src in zip(outs, srcs):
            pltpu.sync_copy(src.at[pl.ds(row0, ROWS_PER_TILE)], slab_v)
            pltpu.sync_copy(slab_v, out_hbm.at[cid, pl.ds(row0, ROWS_PER_TILE)])

    return pl.kernel(
        body,
        out_type=out_type,
        mesh=mesh,
        scratch_types=scratch,
        compiler_params=pltpu.CompilerParams(use_tc_tiling_on_sc=False),
    )


def _tc_call(f, out_shapes):
    return pl.pallas_call(f, out_shape=out_shapes)


def _proj_body(x_ref, w_ref, b_ref, o_ref):
    o_ref[...] = (
        jnp.dot(x_ref[...], w_ref[...], preferred_element_type=jnp.float32)
        + b_ref[...]
    )


def _combine1_body(acc_ref, de_ref, dv_ref, ef_ref, rde_ref, rdv_ref, m_ref):
    de = de_ref[0] + de_ref[1]
    dv = dv_ref[0] + dv_ref[1]
    rde = 1.0 / jnp.maximum(de, 1.0)
    rdv = 1.0 / jnp.maximum(dv, 1.0)
    ef_ref[...] = (acc_ref[0] + acc_ref[1]) * rde
    rde_ref[...] = rde
    rdv_ref[...] = rdv
    m_ref[...] = (dv > 0.0).astype(jnp.float32)


def _scale_leaky_body(acc_ref, rdv_ref, o_ref):
    t = (acc_ref[0] + acc_ref[1]) * rdv_ref[...]
    o_ref[...] = jnp.maximum(t, 0.01 * t)


def _scale_body(acc_ref, rde_ref, o_ref):
    o_ref[...] = (acc_ref[0] + acc_ref[1]) * rde_ref[...]


def _final_body(acc_ref, rdv_ref, m_ref, w2_ref, b2_ref, o_ref):
    n2 = (acc_ref[0] + acc_ref[1]) * rdv_ref[...]
    logits = jnp.dot(
        n2[:N], w2_ref[...], preferred_element_type=jnp.float32
    ) + m_ref[:N, 0:1] * b2_ref[...]
    z = logits - jnp.max(logits, axis=1, keepdims=True)
    o_ref[...] = z - jnp.log(jnp.sum(jnp.exp(z), axis=1, keepdims=True))


def kernel(x, H, W1, b1, W2, b2):
    n_class = W2.shape[1]
    ni = H.shape[1]
    nip = _ceil_to(ni, NW * WIN * 8)
    wpw = nip // (NW * WIN)

    pad = nip - ni
    pad_idx = (N + (jnp.arange(pad, dtype=jnp.int32) % PAD))[None, :]
    Hp = jnp.concatenate([H.astype(jnp.int32), jnp.tile(pad_idx, (2, 1))], axis=1)
    nidx = Hp[0].reshape(nip // WIN, WIN)
    eidx = Hp[1].reshape(nip // WIN, WIN)

    xpad = jnp.pad(x, ((0, P - N), (0, 0)))

    xp = _tc_call(_proj_body, jax.ShapeDtypeStruct((P, D), jnp.float32))(
        xpad, W1, b1.reshape(1, D)
    )

    sc_counts = _make_sc_pass(wpw, with_counts=True)
    sc_plain = _make_sc_pass(wpw, with_counts=False)

    accA, deP, dvP = sc_counts(xp, nidx, eidx)

    ef1, rde, rdv, mask = _tc_call(
        _combine1_body,
        [jax.ShapeDtypeStruct((P, D), jnp.float32)] * 4,
    )(accA, deP, dvP)

    accB = sc_plain(ef1, eidx, nidx)
    if isinstance(accB, (list, tuple)):
        accB = accB[0]

    h = _tc_call(_scale_leaky_body, jax.ShapeDtypeStruct((P, D), jnp.float32))(
        accB, rdv
    )

    accC = sc_plain(h, nidx, eidx)
    if isinstance(accC, (list, tuple)):
        accC = accC[0]

    ef2 = _tc_call(_scale_body, jax.ShapeDtypeStruct((P, D), jnp.float32))(
        accC, rde
    )

    accD = sc_plain(ef2, eidx, nidx)
    if isinstance(accD, (list, tuple)):
        accD = accD[0]

    out = _tc_call(_final_body, jax.ShapeDtypeStruct((N, n_class), jnp.float32))(
        accD, rdv, mask, W2, b2.reshape(1, n_class)
    )
    return out


# speedup vs baseline: 26.7019x; 1.5748x over previous
import jax
import jax.numpy as jnp
from jax import lax
from jax.experimental import pallas as pl
from jax.experimental.pallas import tpu as pltpu
from jax.experimental.pallas import tpu_sc as plsc

N = 10000
PAD = 240
P = N + PAD
D = 16
WIN = 128
NC = 2
NS = 16
NW = NC * NS
RPT = P // NS


def _ceil_to(x, m):
    return (x + m - 1) // m * m


_MESH = plsc.VectorSubcoreMesh(
    core_axis_name="c", subcore_axis_name="s", num_cores=NC, num_subcores=NS
)
_PARAMS = pltpu.CompilerParams(use_tc_tiling_on_sc=False)
_ACC = jax.ShapeDtypeStruct((NC, P, D), jnp.float32)


def _window_pipeline(wpw, table_sp, acc_sp, gidx_v, sidx_v, rows0_v, rows1_v,
                     sem0, sem1, extra_scatter=None):
    def gather(j, buf, sem):
        return pltpu.make_async_copy(table_sp.at[gidx_v.at[j]], buf, sem)

    def scatter(j, buf):
        pltpu.sync_copy(buf, acc_sp.at[sidx_v.at[j]], add=True)
        if extra_scatter is not None:
            extra_scatter(j)

    gather(0, rows0_v, sem0).start()

    def window2(k, _):
        a = 2 * k
        b = a + 1
        gather(b, rows1_v, sem1).start()
        gather(a, rows0_v, sem0).wait()
        scatter(a, rows0_v)

        @pl.when(a + 2 < wpw)
        def _():
            gather(a + 2, rows0_v, sem0).start()

        gather(b, rows1_v, sem1).wait()
        scatter(b, rows1_v)
        return 0
    lax.fori_loop(0, wpw // 2, window2, 0)


def _zero_slab(slab_v, nrows):
    def zrow(i, _):
        slab_v[i, :] = jnp.zeros((D,), jnp.float32)
        return 0
    lax.fori_loop(0, nrows, zrow, 0)


def _copy_out(src_sp, out_hbm, cid, row0, slab_v):
    pltpu.sync_copy(src_sp.at[pl.ds(row0, RPT)], slab_v)
    pltpu.sync_copy(slab_v, out_hbm.at[cid, pl.ds(row0, RPT)])


def _make_sc_first(wpw):
    scratch = [
        pltpu.VMEM_SHARED((P, D), jnp.float32),
        pltpu.VMEM_SHARED((P, D), jnp.float32),
        pltpu.VMEM_SHARED((P, D), jnp.float32),
        pltpu.VMEM_SHARED((P, D), jnp.float32),
        pltpu.VMEM((wpw, WIN), jnp.int32),
        pltpu.VMEM((wpw, WIN), jnp.int32),
        pltpu.VMEM((WIN, D), jnp.float32),
        pltpu.VMEM((WIN, D), jnp.float32),
        pltpu.VMEM((RPT, D), jnp.float32),
        pltpu.VMEM((WIN, D), jnp.float32),
        pltpu.SemaphoreType.DMA,
        pltpu.SemaphoreType.DMA,
    ]

    def body(xp_hbm, gidx_hbm, sidx_hbm, acc_out, de_out, dv_out,
             table_sp, acc_sp, de_sp, dv_sp, gidx_v, sidx_v,
             rows0_v, rows1_v, slab_v, ones_v, sem0, sem1):
        cid = lax.axis_index("c")
        sid = lax.axis_index("s")
        wid = cid * NS + sid
        row0 = sid * RPT

        _zero_slab(slab_v, RPT)
        pltpu.sync_copy(slab_v, acc_sp.at[pl.ds(row0, RPT)])
        pltpu.sync_copy(slab_v, de_sp.at[pl.ds(row0, RPT)])
        pltpu.sync_copy(slab_v, dv_sp.at[pl.ds(row0, RPT)])

        def orow(i, _):
            ones_v[i, :] = jnp.ones((D,), jnp.float32)
            return 0
        lax.fori_loop(0, WIN, orow, 0)

        pltpu.sync_copy(xp_hbm.at[pl.ds(row0, RPT)], slab_v)
        pltpu.sync_copy(slab_v, table_sp.at[pl.ds(row0, RPT)])

        base = wid * wpw
        pltpu.sync_copy(gidx_hbm.at[pl.ds(base, wpw)], gidx_v)
        pltpu.sync_copy(sidx_hbm.at[pl.ds(base, wpw)], sidx_v)

        plsc.subcore_barrier()

        def counts(j):
            pltpu.sync_copy(ones_v, de_sp.at[sidx_v.at[j]], add=True)
            pltpu.sync_copy(ones_v, dv_sp.at[gidx_v.at[j]], add=True)

        _window_pipeline(wpw, table_sp, acc_sp, gidx_v, sidx_v,
                         rows0_v, rows1_v, sem0, sem1, extra_scatter=counts)

        plsc.subcore_barrier()

        _zero_slab(slab_v, 0)
        _copy_out(acc_sp, acc_out, cid, row0, slab_v)
        _copy_out(de_sp, de_out, cid, row0, slab_v)
        _copy_out(dv_sp, dv_out, cid, row0, slab_v)

    return pl.kernel(body, out_type=[_ACC, _ACC, _ACC], mesh=_MESH,
                     scratch_types=scratch, compiler_params=_PARAMS)


def _make_sc_mid(wpw, leaky):
    scratch = [
        pltpu.VMEM_SHARED((P, D), jnp.float32),
        pltpu.VMEM_SHARED((P, D), jnp.float32),
        pltpu.VMEM((wpw, WIN), jnp.int32),
        pltpu.VMEM((wpw, WIN), jnp.int32),
        pltpu.VMEM((WIN, D), jnp.float32),
        pltpu.VMEM((WIN, D), jnp.float32),
        pltpu.VMEM((RPT, D), jnp.float32),
        pltpu.VMEM((2, RPT, D), jnp.float32),
        pltpu.VMEM((2, RPT, D), jnp.float32),
        pltpu.VMEM((RPT, D), jnp.float32),
        pltpu.SemaphoreType.DMA,
        pltpu.SemaphoreType.DMA,
    ]

    def body(accin_hbm, cnt_hbm, gidx_hbm, sidx_hbm, acc_out,
             table_sp, acc_sp, gidx_v, sidx_v, rows0_v, rows1_v,
             slab_v, a_v, d_v, t_v, sem0, sem1):
        cid = lax.axis_index("c")
        sid = lax.axis_index("s")
        wid = cid * NS + sid
        row0 = sid * RPT

        _zero_slab(slab_v, RPT)
        pltpu.sync_copy(slab_v, acc_sp.at[pl.ds(row0, RPT)])

        pltpu.sync_copy(accin_hbm.at[0, pl.ds(row0, RPT)], a_v.at[0])
        pltpu.sync_copy(accin_hbm.at[1, pl.ds(row0, RPT)], a_v.at[1])
        pltpu.sync_copy(cnt_hbm.at[0, pl.ds(row0, RPT)], d_v.at[0])
        pltpu.sync_copy(cnt_hbm.at[1, pl.ds(row0, RPT)], d_v.at[1])

        def crow(i, _):
            s = a_v[0, i, :] + a_v[1, i, :]
            cnt = d_v[0, i, :] + d_v[1, i, :]
            t = s * (1.0 / jnp.maximum(cnt, 1.0))
            if leaky:
                t = jnp.maximum(t, 0.01 * t)
            t_v[i, :] = t
            return 0
        lax.fori_loop(0, RPT, crow, 0)
        pltpu.sync_copy(t_v, table_sp.at[pl.ds(row0, RPT)])

        base = wid * wpw
        pltpu.sync_copy(gidx_hbm.at[pl.ds(base, wpw)], gidx_v)
        pltpu.sync_copy(sidx_hbm.at[pl.ds(base, wpw)], sidx_v)

        plsc.subcore_barrier()

        _window_pipeline(wpw, table_sp, acc_sp, gidx_v, sidx_v,
                         rows0_v, rows1_v, sem0, sem1)

        plsc.subcore_barrier()

        _copy_out(acc_sp, acc_out, cid, row0, slab_v)

    return pl.kernel(body, out_type=[_ACC], mesh=_MESH,
                     scratch_types=scratch, compiler_params=_PARAMS)


def _proj_body(x_ref, w_ref, b_ref, o_ref):
    o_ref[...] = (
        jnp.dot(x_ref[...], w_ref[...], preferred_element_type=jnp.float32)
        + b_ref[...]
    )


def _final_body(acc_ref, dv_ref, w2_ref, b2_ref, o_ref):
    dv = dv_ref[0] + dv_ref[1]
    n2 = (acc_ref[0] + acc_ref[1]) * (1.0 / jnp.maximum(dv, 1.0))
    mask = (dv > 0.0).astype(jnp.float32)
    logits = jnp.dot(
        n2[:N], w2_ref[...], preferred_element_type=jnp.float32
    ) + mask[:N, 0:1] * b2_ref[...]
    z = logits - jnp.max(logits, axis=1, keepdims=True)
    o_ref[...] = z - jnp.log(jnp.sum(jnp.exp(z), axis=1, keepdims=True))


def kernel(x, H, W1, b1, W2, b2):
    n_class = W2.shape[1]
    ni = H.shape[1]
    nip = _ceil_to(ni, NW * WIN * 8)
    wpw = nip // (NW * WIN)

    pad = nip - ni
    pad_idx = (N + (jnp.arange(pad, dtype=jnp.int32) % PAD))[None, :]
    Hp = jnp.concatenate([H.astype(jnp.int32), jnp.tile(pad_idx, (2, 1))], axis=1)
    nidx = Hp[0].reshape(nip // WIN, WIN)
    eidx = Hp[1].reshape(nip // WIN, WIN)

    xpad = jnp.pad(x, ((0, P - N), (0, 0)))

    xp = pl.pallas_call(
        _proj_body, out_shape=jax.ShapeDtypeStruct((P, D), jnp.float32)
    )(xpad, W1, b1.reshape(1, D))

    sc_first = _make_sc_first(wpw)
    sc_mid = _make_sc_mid(wpw, leaky=False)
    sc_mid_leaky = _make_sc_mid(wpw, leaky=True)

    accA, deP, dvP = sc_first(xp, nidx, eidx)
    (accB,) = sc_mid(accA, deP, eidx, nidx)
    (accC,) = sc_mid_leaky(accB, dvP, nidx, eidx)
    (accD,) = sc_mid(accC, deP, eidx, nidx)

    out = pl.pallas_call(
        _final_body, out_shape=jax.ShapeDtypeStruct((N, n_class), jnp.float32)
    )(accD, dvP, W2, b2.reshape(1, n_class))
    return out


# speedup vs baseline: 27.7432x; 1.0390x over previous
import jax
import jax.numpy as jnp
from jax import lax
from jax.experimental import pallas as pl
from jax.experimental.pallas import tpu as pltpu
from jax.experimental.pallas import tpu_sc as plsc

N = 10000
PAD = 240
P = N + PAD
D = 16
WIN = 128
NC = 2
NS = 16
NW = NC * NS
RPT = P // NS


def _ceil_to(x, m):
    return (x + m - 1) // m * m


_MESH = plsc.VectorSubcoreMesh(
    core_axis_name="c", subcore_axis_name="s", num_cores=NC, num_subcores=NS
)
_PARAMS = pltpu.CompilerParams(use_tc_tiling_on_sc=False)
_ACC = jax.ShapeDtypeStruct((NC, P, D), jnp.float32)


def _window_pipeline(my_w, table_sp, acc_sp, gidx_v, sidx_v, rows0_v, rows1_v,
                     sem0, sem1, extra_scatter=None):
    def gather(j, buf, sem):
        return pltpu.make_async_copy(table_sp.at[gidx_v.at[j]], buf, sem)

    def scatter(j, buf):
        pltpu.sync_copy(buf, acc_sp.at[sidx_v.at[j]], add=True)
        if extra_scatter is not None:
            extra_scatter(j)

    gather(0, rows0_v, sem0).start()

    def window2(k, _):
        a = 2 * k
        b = a + 1
        gather(b, rows1_v, sem1).start()
        gather(a, rows0_v, sem0).wait()
        scatter(a, rows0_v)

        @pl.when(a + 2 < my_w)
        def _():
            gather(a + 2, rows0_v, sem0).start()

        gather(b, rows1_v, sem1).wait()
        scatter(b, rows1_v)
        return 0
    lax.fori_loop(0, my_w // 2, window2, 0)


def _zero_slab(slab_v, nrows):
    def zrow(i, _):
        slab_v[i, :] = jnp.zeros((D,), jnp.float32)
        return 0
    lax.fori_loop(0, nrows, zrow, 0)


def _copy_out(src_sp, out_hbm, cid, row0, slab_v):
    pltpu.sync_copy(src_sp.at[pl.ds(row0, RPT)], slab_v)
    pltpu.sync_copy(slab_v, out_hbm.at[cid, pl.ds(row0, RPT)])


def _stage_indices(h3_hbm, grow, srow, gidx_v, sidx_v, wid, wpw, tw):
    fw = tw // wpw
    tailw = tw - fw * wpw
    base = wid * wpw
    if tailw == 0:
        pltpu.sync_copy(h3_hbm.at[grow, pl.ds(base, wpw)], gidx_v)
        pltpu.sync_copy(h3_hbm.at[srow, pl.ds(base, wpw)], sidx_v)
        return wpw

    @pl.when(wid < fw)
    def _():
        pltpu.sync_copy(h3_hbm.at[grow, pl.ds(base, wpw)], gidx_v)
        pltpu.sync_copy(h3_hbm.at[srow, pl.ds(base, wpw)], sidx_v)

    @pl.when(wid >= fw)
    def _():
        pltpu.sync_copy(
            h3_hbm.at[grow, pl.ds(fw * wpw, tailw)], gidx_v.at[pl.ds(0, tailw)]
        )
        pltpu.sync_copy(
            h3_hbm.at[srow, pl.ds(fw * wpw, tailw)], sidx_v.at[pl.ds(0, tailw)]
        )

    return jnp.where(wid < fw, wpw, tailw)


def _make_sc_first(wpw, tw, grow, srow):
    scratch = [
        pltpu.VMEM_SHARED((P, D), jnp.float32),
        pltpu.VMEM_SHARED((P, D), jnp.float32),
        pltpu.VMEM_SHARED((P, D), jnp.float32),
        pltpu.VMEM_SHARED((P, D), jnp.float32),
        pltpu.VMEM((wpw, WIN), jnp.int32),
        pltpu.VMEM((wpw, WIN), jnp.int32),
        pltpu.VMEM((WIN, D), jnp.float32),
        pltpu.VMEM((WIN, D), jnp.float32),
        pltpu.VMEM((RPT, D), jnp.float32),
        pltpu.VMEM((WIN, D), jnp.float32),
        pltpu.SemaphoreType.DMA,
        pltpu.SemaphoreType.DMA,
    ]

    def body(xp_hbm, h3_hbm, acc_out, de_out, dv_out,
             table_sp, acc_sp, de_sp, dv_sp, gidx_v, sidx_v,
             rows0_v, rows1_v, slab_v, ones_v, sem0, sem1):
        cid = lax.axis_index("c")
        sid = lax.axis_index("s")
        wid = cid * NS + sid
        row0 = sid * RPT

        _zero_slab(slab_v, RPT)
        pltpu.sync_copy(slab_v, acc_sp.at[pl.ds(row0, RPT)])
        pltpu.sync_copy(slab_v, de_sp.at[pl.ds(row0, RPT)])
        pltpu.sync_copy(slab_v, dv_sp.at[pl.ds(row0, RPT)])

        def orow(i, _):
            ones_v[i, :] = jnp.ones((D,), jnp.float32)
            return 0
        lax.fori_loop(0, WIN, orow, 0)

        pltpu.sync_copy(xp_hbm.at[pl.ds(row0, RPT)], slab_v)
        pltpu.sync_copy(slab_v, table_sp.at[pl.ds(row0, RPT)])

        my_w = _stage_indices(h3_hbm, grow, srow, gidx_v, sidx_v, wid, wpw, tw)

        plsc.subcore_barrier()

        def counts(j):
            pltpu.sync_copy(ones_v, de_sp.at[sidx_v.at[j]], add=True)
            pltpu.sync_copy(ones_v, dv_sp.at[gidx_v.at[j]], add=True)

        _window_pipeline(my_w, table_sp, acc_sp, gidx_v, sidx_v,
                         rows0_v, rows1_v, sem0, sem1, extra_scatter=counts)

        plsc.subcore_barrier()

        _copy_out(acc_sp, acc_out, cid, row0, slab_v)
        _copy_out(de_sp, de_out, cid, row0, slab_v)
        _copy_out(dv_sp, dv_out, cid, row0, slab_v)

    return pl.kernel(body, out_type=[_ACC, _ACC, _ACC], mesh=_MESH,
                     scratch_types=scratch, compiler_params=_PARAMS)


def _make_sc_mid(wpw, tw, grow, srow, leaky):
    scratch = [
        pltpu.VMEM_SHARED((P, D), jnp.float32),
        pltpu.VMEM_SHARED((P, D), jnp.float32),
        pltpu.VMEM((wpw, WIN), jnp.int32),
        pltpu.VMEM((wpw, WIN), jnp.int32),
        pltpu.VMEM((WIN, D), jnp.float32),
        pltpu.VMEM((WIN, D), jnp.float32),
        pltpu.VMEM((RPT, D), jnp.float32),
        pltpu.VMEM((2, RPT, D), jnp.float32),
        pltpu.VMEM((2, RPT, D), jnp.float32),
        pltpu.VMEM((RPT, D), jnp.float32),
        pltpu.SemaphoreType.DMA,
        pltpu.SemaphoreType.DMA,
    ]

    def body(accin_hbm, cnt_hbm, h3_hbm, acc_out,
             table_sp, acc_sp, gidx_v, sidx_v, rows0_v, rows1_v,
             slab_v, a_v, d_v, t_v, sem0, sem1):
        cid = lax.axis_index("c")
        sid = lax.axis_index("s")
        wid = cid * NS + sid
        row0 = sid * RPT

        _zero_slab(slab_v, RPT)
        pltpu.sync_copy(slab_v, acc_sp.at[pl.ds(row0, RPT)])

        pltpu.sync_copy(accin_hbm.at[0, pl.ds(row0, RPT)], a_v.at[0])
        pltpu.sync_copy(accin_hbm.at[1, pl.ds(row0, RPT)], a_v.at[1])
        pltpu.sync_copy(cnt_hbm.at[0, pl.ds(row0, RPT)], d_v.at[0])
        pltpu.sync_copy(cnt_hbm.at[1, pl.ds(row0, RPT)], d_v.at[1])

        def crow(i, _):
            s = a_v[0, i, :] + a_v[1, i, :]
            cnt = d_v[0, i, :] + d_v[1, i, :]
            t = s * (1.0 / jnp.maximum(cnt, 1.0))
            if leaky:
                t = jnp.maximum(t, 0.01 * t)
            t_v[i, :] = t
            return 0
        lax.fori_loop(0, RPT, crow, 0)
        pltpu.sync_copy(t_v, table_sp.at[pl.ds(row0, RPT)])

        my_w = _stage_indices(h3_hbm, grow, srow, gidx_v, sidx_v, wid, wpw, tw)

        plsc.subcore_barrier()

        _window_pipeline(my_w, table_sp, acc_sp, gidx_v, sidx_v,
                         rows0_v, rows1_v, sem0, sem1)

        plsc.subcore_barrier()

        _copy_out(acc_sp, acc_out, cid, row0, slab_v)

    return pl.kernel(body, out_type=[_ACC], mesh=_MESH,
                     scratch_types=scratch, compiler_params=_PARAMS)


def _proj_body(x_ref, w_ref, b_ref, o_ref):
    o_ref[0:N, :] = (
        jnp.dot(x_ref[...], w_ref[...], preferred_element_type=jnp.float32)
        + b_ref[...]
    )
    o_ref[N:P, :] = jnp.zeros((P - N, D), jnp.float32)


def _final_body(acc_ref, dv_ref, w2_ref, b2_ref, o_ref):
    dv = dv_ref[0] + dv_ref[1]
    n2 = (acc_ref[0] + acc_ref[1]) * (1.0 / jnp.maximum(dv, 1.0))
    mask = (dv > 0.0).astype(jnp.float32)
    logits = jnp.dot(
        n2[:N], w2_ref[...], preferred_element_type=jnp.float32
    ) + mask[:N, 0:1] * b2_ref[...]
    z = logits - jnp.max(logits, axis=1, keepdims=True)
    o_ref[...] = z - jnp.log(jnp.sum(jnp.exp(z), axis=1, keepdims=True))


def kernel(x, H, W1, b1, W2, b2):
    n_class = W2.shape[1]
    ni = H.shape[1]
    Hi = H.astype(jnp.int32)
    if ni % WIN:
        padn = WIN - ni % WIN
        Hi = jnp.concatenate(
            [Hi, jnp.full((2, padn), N, jnp.int32)], axis=1
        )
        ni += padn
    tw = ni // WIN
    wpw = _ceil_to(_ceil_to(ni, NW * WIN) // (NW * WIN), 8)
    h3 = Hi.reshape(2, tw, WIN)

    xp = pl.pallas_call(
        _proj_body, out_shape=jax.ShapeDtypeStruct((P, D), jnp.float32)
    )(x, W1, b1.reshape(1, D))

    sc_first = _make_sc_first(wpw, tw, grow=0, srow=1)
    sc_mid = _make_sc_mid(wpw, tw, grow=1, srow=0, leaky=False)
    sc_mid_leaky = _make_sc_mid(wpw, tw, grow=0, srow=1, leaky=True)

    accA, deP, dvP = sc_first(xp, h3)
    (accB,) = sc_mid(accA, deP, h3)
    (accC,) = sc_mid_leaky(accB, dvP, h3)
    (accD,) = sc_mid(accC, deP, h3)

    out = pl.pallas_call(
        _final_body, out_shape=jax.ShapeDtypeStruct((N, n_class), jnp.float32)
    )(accD, dvP, W2, b2.reshape(1, n_class))
    return out


# speedup vs baseline: 29.5815x; 1.0663x over previous
import jax
import jax.numpy as jnp
from jax import lax
from jax.experimental import pallas as pl
from jax.experimental.pallas import tpu as pltpu
from jax.experimental.pallas import tpu_sc as plsc

N = 10000
PAD = 240
P = N + PAD
D = 16
WIN = 128
NC = 2
NS = 16
NW = NC * NS
RPT = P // NS


def _ceil_to(x, m):
    return (x + m - 1) // m * m


_MESH = plsc.VectorSubcoreMesh(
    core_axis_name="c", subcore_axis_name="s", num_cores=NC, num_subcores=NS
)
_PARAMS = pltpu.CompilerParams(use_tc_tiling_on_sc=False)
_ACC = jax.ShapeDtypeStruct((NC, P, D), jnp.float32)


def _window_pipeline(my_w, table_sp, acc_sp, gidx_v, sidx_v, rows0_v, rows1_v,
                     sem0, sem1, extra_scatter=None):
    def gather(j, buf, sem):
        return pltpu.make_async_copy(table_sp.at[gidx_v.at[j]], buf, sem)

    def scatter(j, buf):
        pltpu.sync_copy(buf, acc_sp.at[sidx_v.at[j]], add=True)
        if extra_scatter is not None:
            extra_scatter(j)

    gather(0, rows0_v, sem0).start()

    def window2(k, _):
        a = 2 * k
        b = a + 1
        gather(b, rows1_v, sem1).start()
        gather(a, rows0_v, sem0).wait()
        scatter(a, rows0_v)

        @pl.when(a + 2 < my_w)
        def _():
            gather(a + 2, rows0_v, sem0).start()

        gather(b, rows1_v, sem1).wait()
        scatter(b, rows1_v)
        return 0
    lax.fori_loop(0, my_w // 2, window2, 0)


def _zero_slab(slab_v, nrows):
    def zrow(i, _):
        slab_v[i, :] = jnp.zeros((D,), jnp.float32)
        return 0
    lax.fori_loop(0, nrows, zrow, 0)


def _copy_out(src_sp, out_hbm, cid, row0, slab_v):
    pltpu.sync_copy(src_sp.at[pl.ds(row0, RPT)], slab_v)
    pltpu.sync_copy(slab_v, out_hbm.at[cid, pl.ds(row0, RPT)])


def _stage_indices(hw_hbm, goff, soff, gidx_v, sidx_v, wid, wpw, tw):
    fw = tw // wpw
    tailw = tw - fw * wpw
    base = wid * wpw
    if tailw == 0:
        pltpu.sync_copy(hw_hbm.at[pl.ds(goff + base, wpw)], gidx_v)
        pltpu.sync_copy(hw_hbm.at[pl.ds(soff + base, wpw)], sidx_v)
        return wpw

    @pl.when(wid < fw)
    def _():
        pltpu.sync_copy(hw_hbm.at[pl.ds(goff + base, wpw)], gidx_v)
        pltpu.sync_copy(hw_hbm.at[pl.ds(soff + base, wpw)], sidx_v)

    @pl.when(wid >= fw)
    def _():
        pltpu.sync_copy(
            hw_hbm.at[pl.ds(goff + fw * wpw, tailw)],
            gidx_v.at[pl.ds(0, tailw)],
        )
        pltpu.sync_copy(
            hw_hbm.at[pl.ds(soff + fw * wpw, tailw)],
            sidx_v.at[pl.ds(0, tailw)],
        )

    return jnp.where(wid < fw, wpw, tailw)


def _make_sc_first(wpw, tw, grow, srow):
    scratch = [
        pltpu.VMEM_SHARED((P, D), jnp.float32),
        pltpu.VMEM_SHARED((P, D), jnp.float32),
        pltpu.VMEM_SHARED((P, D), jnp.float32),
        pltpu.VMEM_SHARED((P, D), jnp.float32),
        pltpu.VMEM((wpw, WIN), jnp.int32),
        pltpu.VMEM((wpw, WIN), jnp.int32),
        pltpu.VMEM((WIN, D), jnp.float32),
        pltpu.VMEM((WIN, D), jnp.float32),
        pltpu.VMEM((RPT, D), jnp.float32),
        pltpu.VMEM((WIN, D), jnp.float32),
        pltpu.SemaphoreType.DMA,
        pltpu.SemaphoreType.DMA,
    ]

    def body(xp_hbm, h3_hbm, acc_out, de_out, dv_out,
             table_sp, acc_sp, de_sp, dv_sp, gidx_v, sidx_v,
             rows0_v, rows1_v, slab_v, ones_v, sem0, sem1):
        cid = lax.axis_index("c")
        sid = lax.axis_index("s")
        wid = cid * NS + sid
        row0 = sid * RPT

        _zero_slab(slab_v, RPT)
        pltpu.sync_copy(slab_v, acc_sp.at[pl.ds(row0, RPT)])
        pltpu.sync_copy(slab_v, de_sp.at[pl.ds(row0, RPT)])
        pltpu.sync_copy(slab_v, dv_sp.at[pl.ds(row0, RPT)])

        def orow(i, _):
            ones_v[i, :] = jnp.ones((D,), jnp.float32)
            return 0
        lax.fori_loop(0, WIN, orow, 0)

        pltpu.sync_copy(xp_hbm.at[pl.ds(row0, RPT)], slab_v)
        pltpu.sync_copy(slab_v, table_sp.at[pl.ds(row0, RPT)])

        my_w = _stage_indices(h3_hbm, grow, srow, gidx_v, sidx_v, wid, wpw, tw)

        plsc.subcore_barrier()

        def counts(j):
            pltpu.sync_copy(ones_v, de_sp.at[sidx_v.at[j]], add=True)
            pltpu.sync_copy(ones_v, dv_sp.at[gidx_v.at[j]], add=True)

        _window_pipeline(my_w, table_sp, acc_sp, gidx_v, sidx_v,
                         rows0_v, rows1_v, sem0, sem1, extra_scatter=counts)

        plsc.subcore_barrier()

        _copy_out(acc_sp, acc_out, cid, row0, slab_v)
        _copy_out(de_sp, de_out, cid, row0, slab_v)
        _copy_out(dv_sp, dv_out, cid, row0, slab_v)

    return pl.kernel(body, out_type=[_ACC, _ACC, _ACC], mesh=_MESH,
                     scratch_types=scratch, compiler_params=_PARAMS)


def _make_sc_mid(wpw, tw, grow, srow, leaky):
    scratch = [
        pltpu.VMEM_SHARED((P, D), jnp.float32),
        pltpu.VMEM_SHARED((P, D), jnp.float32),
        pltpu.VMEM((wpw, WIN), jnp.int32),
        pltpu.VMEM((wpw, WIN), jnp.int32),
        pltpu.VMEM((WIN, D), jnp.float32),
        pltpu.VMEM((WIN, D), jnp.float32),
        pltpu.VMEM((RPT, D), jnp.float32),
        pltpu.VMEM((2, RPT, D), jnp.float32),
        pltpu.VMEM((2, RPT, D), jnp.float32),
        pltpu.VMEM((RPT, D), jnp.float32),
        pltpu.SemaphoreType.DMA,
        pltpu.SemaphoreType.DMA,
    ]

    def body(accin_hbm, cnt_hbm, h3_hbm, acc_out,
             table_sp, acc_sp, gidx_v, sidx_v, rows0_v, rows1_v,
             slab_v, a_v, d_v, t_v, sem0, sem1):
        cid = lax.axis_index("c")
        sid = lax.axis_index("s")
        wid = cid * NS + sid
        row0 = sid * RPT

        _zero_slab(slab_v, RPT)
        pltpu.sync_copy(slab_v, acc_sp.at[pl.ds(row0, RPT)])

        pltpu.sync_copy(accin_hbm.at[0, pl.ds(row0, RPT)], a_v.at[0])
        pltpu.sync_copy(accin_hbm.at[1, pl.ds(row0, RPT)], a_v.at[1])
        pltpu.sync_copy(cnt_hbm.at[0, pl.ds(row0, RPT)], d_v.at[0])
        pltpu.sync_copy(cnt_hbm.at[1, pl.ds(row0, RPT)], d_v.at[1])

        def crow(i, _):
            s = a_v[0, i, :] + a_v[1, i, :]
            cnt = d_v[0, i, :] + d_v[1, i, :]
            t = s * (1.0 / jnp.maximum(cnt, 1.0))
            if leaky:
                t = jnp.maximum(t, 0.01 * t)
            t_v[i, :] = t
            return 0
        lax.fori_loop(0, RPT, crow, 0)
        pltpu.sync_copy(t_v, table_sp.at[pl.ds(row0, RPT)])

        my_w = _stage_indices(h3_hbm, grow, srow, gidx_v, sidx_v, wid, wpw, tw)

        plsc.subcore_barrier()

        _window_pipeline(my_w, table_sp, acc_sp, gidx_v, sidx_v,
                         rows0_v, rows1_v, sem0, sem1)

        plsc.subcore_barrier()

        _copy_out(acc_sp, acc_out, cid, row0, slab_v)

    return pl.kernel(body, out_type=[_ACC], mesh=_MESH,
                     scratch_types=scratch, compiler_params=_PARAMS)


def _proj_body(x_ref, w_ref, b_ref, o_ref):
    o_ref[0:N, :] = (
        jnp.dot(x_ref[...], w_ref[...], preferred_element_type=jnp.float32)
        + b_ref[...]
    )
    o_ref[N:P, :] = jnp.zeros((P - N, D), jnp.float32)


def _final_body(acc_ref, dv_ref, w_ref, g_ref, o_ref):
    dv = dv_ref[0] + dv_ref[1]
    n2 = (acc_ref[0] + acc_ref[1]) * (1.0 / jnp.maximum(dv, 1.0))
    mask = (dv > 0.0).astype(jnp.float32)
    xa = jnp.concatenate([n2, mask], axis=1)
    logits = jnp.dot(xa, w_ref[...], preferred_element_type=jnp.float32)
    z = logits - jnp.max(logits, axis=1, keepdims=True)
    s = jnp.dot(jnp.exp(z), g_ref[...], preferred_element_type=jnp.float32)
    o_ref[...] = z - jnp.log(s)


def kernel(x, H, W1, b1, W2, b2):
    n_class = W2.shape[1]
    ni = H.shape[1]
    Hi = H.astype(jnp.int32)
    if ni % WIN:
        padn = WIN - ni % WIN
        Hi = jnp.concatenate(
            [Hi, jnp.full((2, padn), N, jnp.int32)], axis=1
        )
        ni += padn
    tw = ni // WIN
    wpw = _ceil_to(_ceil_to(ni, NW * WIN) // (NW * WIN), 8)
    hw = Hi.reshape(2 * tw, WIN)

    xp = pl.pallas_call(
        _proj_body, out_shape=jax.ShapeDtypeStruct((P, D), jnp.float32)
    )(x, W1, b1.reshape(1, D))

    sc_first = _make_sc_first(wpw, tw, grow=0, srow=tw)
    sc_mid = _make_sc_mid(wpw, tw, grow=tw, srow=0, leaky=False)
    sc_mid_leaky = _make_sc_mid(wpw, tw, grow=0, srow=tw, leaky=True)

    accA, deP, dvP = sc_first(xp, hw)
    (accB,) = sc_mid(accA, deP, hw)
    (accC,) = sc_mid_leaky(accB, dvP, hw)
    (accD,) = sc_mid(accC, deP, hw)

    p8 = P // 8
    eye8 = jnp.eye(8, dtype=jnp.float32)
    w_aug = jnp.concatenate(
        [
            jnp.kron(eye8, W2),
            jnp.kron(
                eye8,
                jnp.concatenate(
                    [b2.reshape(1, n_class),
                     jnp.zeros((D - 1, n_class), jnp.float32)]
                ),
            ),
        ],
        axis=0,
    )
    g_sum = jnp.kron(eye8, jnp.ones((n_class, n_class), jnp.float32))
    out8 = pl.pallas_call(
        _final_body,
        out_shape=jax.ShapeDtypeStruct((p8, 8 * n_class), jnp.float32),
    )(accD.reshape(NC, p8, 8 * D), dvP.reshape(NC, p8, 8 * D), w_aug, g_sum)
    return out8.reshape(P, n_class)[:N]


# speedup vs baseline: 31.3757x; 1.0607x over previous
import jax
import jax.numpy as jnp
from jax import lax
from jax.experimental import pallas as pl
from jax.experimental.pallas import tpu as pltpu
from jax.experimental.pallas import tpu_sc as plsc

N = 10000
PAD = 240
P = N + PAD
D = 16
WIN = 128
NC = 2
NS = 16
NW = NC * NS
RPT = P // NS


def _ceil_to(x, m):
    return (x + m - 1) // m * m


_MESH = plsc.VectorSubcoreMesh(
    core_axis_name="c", subcore_axis_name="s", num_cores=NC, num_subcores=NS
)
_PARAMS = pltpu.CompilerParams(
    use_tc_tiling_on_sc=False, needs_layout_passes=False
)
_ACC = jax.ShapeDtypeStruct((NC, P, D), jnp.float32)


def _window_pipeline(my_w, table_sp, acc_sp, gidx_v, sidx_v, rows0_v, rows1_v,
                     sem0, sem1, extra_scatter=None):
    def gather(j, buf, sem):
        return pltpu.make_async_copy(table_sp.at[gidx_v.at[j]], buf, sem)

    def scatter(j, buf):
        pltpu.sync_copy(buf, acc_sp.at[sidx_v.at[j]], add=True)
        if extra_scatter is not None:
            extra_scatter(j)

    gather(0, rows0_v, sem0).start()

    def window2(k, _):
        a = 2 * k
        b = a + 1
        gather(b, rows1_v, sem1).start()
        gather(a, rows0_v, sem0).wait()
        scatter(a, rows0_v)

        @pl.when(a + 2 < my_w)
        def _():
            gather(a + 2, rows0_v, sem0).start()

        gather(b, rows1_v, sem1).wait()
        scatter(b, rows1_v)
        return 0
    lax.fori_loop(0, my_w // 2, window2, 0)


def _zero_slab(slab_v, nrows):
    def zrow(i, _):
        slab_v[i, :] = jnp.zeros((D,), jnp.float32)
        return 0
    lax.fori_loop(0, nrows, zrow, 0)


def _copy_out(src_sp, out_hbm, cid, row0, slab_v):
    pltpu.sync_copy(src_sp.at[pl.ds(row0, RPT)], slab_v)
    pltpu.sync_copy(slab_v, out_hbm.at[cid, pl.ds(row0, RPT)])


def _stage_indices(hw_hbm, goff, soff, gidx_v, sidx_v, wid, wpw, tw):
    fw = tw // wpw
    tailw = tw - fw * wpw
    base = wid * wpw
    if tailw == 0:
        pltpu.sync_copy(hw_hbm.at[pl.ds(goff + base, wpw)], gidx_v)
        pltpu.sync_copy(hw_hbm.at[pl.ds(soff + base, wpw)], sidx_v)
        return wpw

    @pl.when(wid < fw)
    def _():
        pltpu.sync_copy(hw_hbm.at[pl.ds(goff + base, wpw)], gidx_v)
        pltpu.sync_copy(hw_hbm.at[pl.ds(soff + base, wpw)], sidx_v)

    @pl.when(wid >= fw)
    def _():
        pltpu.sync_copy(
            hw_hbm.at[pl.ds(goff + fw * wpw, tailw)],
            gidx_v.at[pl.ds(0, tailw)],
        )
        pltpu.sync_copy(
            hw_hbm.at[pl.ds(soff + fw * wpw, tailw)],
            sidx_v.at[pl.ds(0, tailw)],
        )

    return jnp.where(wid < fw, wpw, tailw)


def _make_sc_first(wpw, tw, grow, srow):
    scratch = [
        pltpu.VMEM_SHARED((P, D), jnp.float32),
        pltpu.VMEM_SHARED((P, D), jnp.float32),
        pltpu.VMEM_SHARED((P,), jnp.float32),
        pltpu.VMEM_SHARED((P,), jnp.float32),
        pltpu.VMEM((wpw, WIN), jnp.int32),
        pltpu.VMEM((wpw, WIN), jnp.int32),
        pltpu.VMEM((WIN, D), jnp.float32),
        pltpu.VMEM((WIN, D), jnp.float32),
        pltpu.VMEM((RPT, D), jnp.float32),
        pltpu.VMEM((WIN,), jnp.float32),
        pltpu.VMEM((RPT,), jnp.float32),
        pltpu.SemaphoreType.DMA,
        pltpu.SemaphoreType.DMA,
    ]
    cnt_out = jax.ShapeDtypeStruct((NC, P), jnp.float32)

    def body(xp_hbm, h3_hbm, acc_out, de_out, dv_out,
             table_sp, acc_sp, de_sp, dv_sp, gidx_v, sidx_v,
             rows0_v, rows1_v, slab_v, ones_v, cbuf_v, sem0, sem1):
        cid = lax.axis_index("c")
        sid = lax.axis_index("s")
        wid = cid * NS + sid
        row0 = sid * RPT

        _zero_slab(slab_v, RPT)
        pltpu.sync_copy(slab_v, acc_sp.at[pl.ds(row0, RPT)])

        def zc(i, _):
            cbuf_v[pl.ds(i * D, D)] = jnp.zeros((D,), jnp.float32)
            return 0
        lax.fori_loop(0, RPT // D, zc, 0)
        pltpu.sync_copy(cbuf_v, de_sp.at[pl.ds(row0, RPT)])
        pltpu.sync_copy(cbuf_v, dv_sp.at[pl.ds(row0, RPT)])

        def orow(i, _):
            ones_v[pl.ds(i * D, D)] = jnp.ones((D,), jnp.float32)
            return 0
        lax.fori_loop(0, WIN // D, orow, 0)

        pltpu.sync_copy(xp_hbm.at[pl.ds(row0, RPT)], slab_v)
        pltpu.sync_copy(slab_v, table_sp.at[pl.ds(row0, RPT)])

        my_w = _stage_indices(h3_hbm, grow, srow, gidx_v, sidx_v, wid, wpw, tw)

        plsc.subcore_barrier()

        def counts(j):
            pltpu.sync_copy(ones_v, de_sp.at[sidx_v.at[j]], add=True)
            pltpu.sync_copy(ones_v, dv_sp.at[gidx_v.at[j]], add=True)

        _window_pipeline(my_w, table_sp, acc_sp, gidx_v, sidx_v,
                         rows0_v, rows1_v, sem0, sem1, extra_scatter=counts)

        plsc.subcore_barrier()

        _copy_out(acc_sp, acc_out, cid, row0, slab_v)
        pltpu.sync_copy(de_sp.at[pl.ds(row0, RPT)], cbuf_v)
        pltpu.sync_copy(cbuf_v, de_out.at[cid, pl.ds(row0, RPT)])
        pltpu.sync_copy(dv_sp.at[pl.ds(row0, RPT)], cbuf_v)
        pltpu.sync_copy(cbuf_v, dv_out.at[cid, pl.ds(row0, RPT)])

    return pl.kernel(body, out_type=[_ACC, cnt_out, cnt_out], mesh=_MESH,
                     scratch_types=scratch, compiler_params=_PARAMS)


def _make_sc_mid(wpw, tw, grow, srow, leaky):
    scratch = [
        pltpu.VMEM_SHARED((P, D), jnp.float32),
        pltpu.VMEM_SHARED((P, D), jnp.float32),
        pltpu.VMEM((wpw, WIN), jnp.int32),
        pltpu.VMEM((wpw, WIN), jnp.int32),
        pltpu.VMEM((WIN, D), jnp.float32),
        pltpu.VMEM((WIN, D), jnp.float32),
        pltpu.VMEM((RPT, D), jnp.float32),
        pltpu.VMEM((2, RPT, D), jnp.float32),
        pltpu.VMEM((2, RPT), jnp.float32),
        pltpu.VMEM((RPT, D), jnp.float32),
        pltpu.SemaphoreType.DMA,
        pltpu.SemaphoreType.DMA,
    ]

    def body(accin_hbm, cnt_hbm, h3_hbm, acc_out,
             table_sp, acc_sp, gidx_v, sidx_v, rows0_v, rows1_v,
             slab_v, a_v, d_v, t_v, sem0, sem1):
        cid = lax.axis_index("c")
        sid = lax.axis_index("s")
        wid = cid * NS + sid
        row0 = sid * RPT

        _zero_slab(slab_v, RPT)
        pltpu.sync_copy(slab_v, acc_sp.at[pl.ds(row0, RPT)])

        pltpu.sync_copy(accin_hbm.at[0, pl.ds(row0, RPT)], a_v.at[0])
        pltpu.sync_copy(accin_hbm.at[1, pl.ds(row0, RPT)], a_v.at[1])
        pltpu.sync_copy(cnt_hbm.at[0, pl.ds(row0, RPT)], d_v.at[0])
        pltpu.sync_copy(cnt_hbm.at[1, pl.ds(row0, RPT)], d_v.at[1])

        lanes = lax.iota(jnp.int32, D)

        def cgroup(g, _):
            c16 = d_v[0, pl.ds(g * D, D)] + d_v[1, pl.ds(g * D, D)]
            r16 = 1.0 / jnp.maximum(c16, 1.0)
            for j in range(D):
                i = g * D + j
                r = jnp.sum(jnp.where(lanes == j, r16, 0.0))
                t = (a_v[0, i, :] + a_v[1, i, :]) * r
                if leaky:
                    t = jnp.maximum(t, 0.01 * t)
                t_v[i, :] = t
            return 0
        lax.fori_loop(0, RPT // D, cgroup, 0)
        pltpu.sync_copy(t_v, table_sp.at[pl.ds(row0, RPT)])

        my_w = _stage_indices(h3_hbm, grow, srow, gidx_v, sidx_v, wid, wpw, tw)

        plsc.subcore_barrier()

        _window_pipeline(my_w, table_sp, acc_sp, gidx_v, sidx_v,
                         rows0_v, rows1_v, sem0, sem1)

        plsc.subcore_barrier()

        _copy_out(acc_sp, acc_out, cid, row0, slab_v)

    return pl.kernel(body, out_type=[_ACC], mesh=_MESH,
                     scratch_types=scratch, compiler_params=_PARAMS)


def _proj_body(x_ref, w_ref, b_ref, o_ref):
    o_ref[0:N, :] = (
        jnp.dot(x_ref[...], w_ref[...], preferred_element_type=jnp.float32)
        + b_ref[...]
    )
    o_ref[N:P, :] = jnp.zeros((P - N, D), jnp.float32)


def _final_body(acc_ref, dv_ref, w_ref, g_ref, o_ref):
    dv = dv_ref[0] + dv_ref[1]
    n2 = (acc_ref[0] + acc_ref[1]) * (1.0 / jnp.maximum(dv, 1.0))
    mask = (dv > 0.0).astype(jnp.float32)
    xa = jnp.concatenate([n2, mask], axis=1)
    logits = jnp.dot(xa, w_ref[...], preferred_element_type=jnp.float32)
    z = logits - jnp.max(logits, axis=1, keepdims=True)
    s = jnp.dot(jnp.exp(z), g_ref[...], preferred_element_type=jnp.float32)
    o_ref[...] = z - jnp.log(s)


def kernel(x, H, W1, b1, W2, b2):
    n_class = W2.shape[1]
    ni = H.shape[1]
    Hi = H.astype(jnp.int32)
    if ni % WIN:
        padn = WIN - ni % WIN
        Hi = jnp.concatenate(
            [Hi, jnp.full((2, padn), N, jnp.int32)], axis=1
        )
        ni += padn
    tw = ni // WIN
    wpw = _ceil_to(_ceil_to(ni, NW * WIN) // (NW * WIN), 8)
    hw = Hi.reshape(2 * tw, WIN)

    xp = pl.pallas_call(
        _proj_body, out_shape=jax.ShapeDtypeStruct((P, D), jnp.float32)
    )(x, W1, b1.reshape(1, D))

    sc_first = _make_sc_first(wpw, tw, grow=0, srow=tw)
    sc_mid = _make_sc_mid(wpw, tw, grow=tw, srow=0, leaky=False)
    sc_mid_leaky = _make_sc_mid(wpw, tw, grow=0, srow=tw, leaky=True)

    accA, deP, dvP = sc_first(xp, hw)
    (accB,) = sc_mid(accA, deP, hw)
    (accC,) = sc_mid_leaky(accB, dvP, hw)
    (accD,) = sc_mid(accC, deP, hw)

    p8 = P // 8
    eye8 = jnp.eye(8, dtype=jnp.float32)
    w_aug = jnp.concatenate(
        [
            jnp.kron(eye8, W2),
            jnp.kron(
                eye8,
                jnp.concatenate(
                    [b2.reshape(1, n_class),
                     jnp.zeros((D - 1, n_class), jnp.float32)]
                ),
            ),
        ],
        axis=0,
    )
    g_sum = jnp.kron(eye8, jnp.ones((n_class, n_class), jnp.float32))
    dv16 = jnp.repeat(dvP, D, axis=-1).reshape(NC, p8, 8 * D)
    out8 = pl.pallas_call(
        _final_body,
        out_shape=jax.ShapeDtypeStruct((p8, 8 * n_class), jnp.float32),
    )(accD.reshape(NC, p8, 8 * D), dv16, w_aug, g_sum)
    return out8.reshape(P, n_class)[:N]


# speedup vs baseline: 32.9517x; 1.0502x over previous
import jax
import jax.numpy as jnp
from jax import lax
from jax.experimental import pallas as pl
from jax.experimental.pallas import tpu as pltpu
from jax.experimental.pallas import tpu_sc as plsc

N = 10000
PAD = 240
P = N + PAD
D = 16
WIN = 256
NC = 2
NS = 16
NW = NC * NS
RPT = P // NS


def _ceil_to(x, m):
    return (x + m - 1) // m * m


_MESH = plsc.VectorSubcoreMesh(
    core_axis_name="c", subcore_axis_name="s", num_cores=NC, num_subcores=NS
)
_PARAMS = pltpu.CompilerParams(
    use_tc_tiling_on_sc=False, needs_layout_passes=False
)
_ACC = jax.ShapeDtypeStruct((NC, P, D), jnp.float32)


def _window_pipeline(my_w, table_sp, acc_sp, gidx_v, sidx_v, rows0_v, rows1_v,
                     sem0, sem1, extra_scatter=None):
    def gather(j, buf, sem):
        return pltpu.make_async_copy(table_sp.at[gidx_v.at[j]], buf, sem)

    def scatter(j, buf):
        pltpu.sync_copy(buf, acc_sp.at[sidx_v.at[j]], add=True)
        if extra_scatter is not None:
            extra_scatter(j)

    gather(0, rows0_v, sem0).start()

    def window2(k, _):
        a = 2 * k
        b = a + 1
        gather(b, rows1_v, sem1).start()
        gather(a, rows0_v, sem0).wait()
        scatter(a, rows0_v)

        @pl.when(a + 2 < my_w)
        def _():
            gather(a + 2, rows0_v, sem0).start()

        gather(b, rows1_v, sem1).wait()
        scatter(b, rows1_v)
        return 0
    lax.fori_loop(0, my_w // 2, window2, 0)


def _zero_slab(slab_v, nrows):
    def zrow(i, _):
        slab_v[i, :] = jnp.zeros((D,), jnp.float32)
        return 0
    lax.fori_loop(0, nrows, zrow, 0)


def _copy_out(src_sp, out_hbm, cid, row0, slab_v):
    pltpu.sync_copy(src_sp.at[pl.ds(row0, RPT)], slab_v)
    pltpu.sync_copy(slab_v, out_hbm.at[cid, pl.ds(row0, RPT)])


def _stage_indices(hw_hbm, goff, soff, gidx_v, sidx_v, wid, wpw, tw):
    fw = tw // wpw
    tailw = tw - fw * wpw
    base = wid * wpw
    if tailw == 0:
        pltpu.sync_copy(hw_hbm.at[pl.ds(goff + base, wpw)], gidx_v)
        pltpu.sync_copy(hw_hbm.at[pl.ds(soff + base, wpw)], sidx_v)
        return wpw

    @pl.when(wid < fw)
    def _():
        pltpu.sync_copy(hw_hbm.at[pl.ds(goff + base, wpw)], gidx_v)
        pltpu.sync_copy(hw_hbm.at[pl.ds(soff + base, wpw)], sidx_v)

    @pl.when(wid >= fw)
    def _():
        pltpu.sync_copy(
            hw_hbm.at[pl.ds(goff + fw * wpw, tailw)],
            gidx_v.at[pl.ds(0, tailw)],
        )
        pltpu.sync_copy(
            hw_hbm.at[pl.ds(soff + fw * wpw, tailw)],
            sidx_v.at[pl.ds(0, tailw)],
        )

    return jnp.where(wid < fw, wpw, tailw)


def _make_sc_first(wpw, tw, grow, srow):
    scratch = [
        pltpu.VMEM_SHARED((P, D), jnp.float32),
        pltpu.VMEM_SHARED((P, D), jnp.float32),
        pltpu.VMEM_SHARED((P,), jnp.float32),
        pltpu.VMEM_SHARED((P,), jnp.float32),
        pltpu.VMEM((wpw, WIN), jnp.int32),
        pltpu.VMEM((wpw, WIN), jnp.int32),
        pltpu.VMEM((WIN, D), jnp.float32),
        pltpu.VMEM((WIN, D), jnp.float32),
        pltpu.VMEM((RPT, D), jnp.float32),
        pltpu.VMEM((WIN,), jnp.float32),
        pltpu.VMEM((RPT,), jnp.float32),
        pltpu.SemaphoreType.DMA,
        pltpu.SemaphoreType.DMA,
    ]
    cnt_out = jax.ShapeDtypeStruct((NC, P), jnp.float32)

    def body(xp_hbm, h3_hbm, acc_out, de_out, dv_out,
             table_sp, acc_sp, de_sp, dv_sp, gidx_v, sidx_v,
             rows0_v, rows1_v, slab_v, ones_v, cbuf_v, sem0, sem1):
        cid = lax.axis_index("c")
        sid = lax.axis_index("s")
        wid = cid * NS + sid
        row0 = sid * RPT

        _zero_slab(slab_v, RPT)
        pltpu.sync_copy(slab_v, acc_sp.at[pl.ds(row0, RPT)])

        def zc(i, _):
            cbuf_v[pl.ds(i * D, D)] = jnp.zeros((D,), jnp.float32)
            return 0
        lax.fori_loop(0, RPT // D, zc, 0)
        pltpu.sync_copy(cbuf_v, de_sp.at[pl.ds(row0, RPT)])
        pltpu.sync_copy(cbuf_v, dv_sp.at[pl.ds(row0, RPT)])

        def orow(i, _):
            ones_v[pl.ds(i * D, D)] = jnp.ones((D,), jnp.float32)
            return 0
        lax.fori_loop(0, WIN // D, orow, 0)

        pltpu.sync_copy(xp_hbm.at[pl.ds(row0, RPT)], slab_v)
        pltpu.sync_copy(slab_v, table_sp.at[pl.ds(row0, RPT)])

        my_w = _stage_indices(h3_hbm, grow, srow, gidx_v, sidx_v, wid, wpw, tw)

        plsc.subcore_barrier()

        def counts(j):
            pltpu.sync_copy(ones_v, de_sp.at[sidx_v.at[j]], add=True)
            pltpu.sync_copy(ones_v, dv_sp.at[gidx_v.at[j]], add=True)

        _window_pipeline(my_w, table_sp, acc_sp, gidx_v, sidx_v,
                         rows0_v, rows1_v, sem0, sem1, extra_scatter=counts)

        plsc.subcore_barrier()

        _copy_out(acc_sp, acc_out, cid, row0, slab_v)
        pltpu.sync_copy(de_sp.at[pl.ds(row0, RPT)], cbuf_v)
        pltpu.sync_copy(cbuf_v, de_out.at[cid, pl.ds(row0, RPT)])
        pltpu.sync_copy(dv_sp.at[pl.ds(row0, RPT)], cbuf_v)
        pltpu.sync_copy(cbuf_v, dv_out.at[cid, pl.ds(row0, RPT)])

    return pl.kernel(body, out_type=[_ACC, cnt_out, cnt_out], mesh=_MESH,
                     scratch_types=scratch, compiler_params=_PARAMS)


def _make_sc_mid(wpw, tw, grow, srow, leaky):
    scratch = [
        pltpu.VMEM_SHARED((P, D), jnp.float32),
        pltpu.VMEM_SHARED((P, D), jnp.float32),
        pltpu.VMEM((wpw, WIN), jnp.int32),
        pltpu.VMEM((wpw, WIN), jnp.int32),
        pltpu.VMEM((WIN, D), jnp.float32),
        pltpu.VMEM((WIN, D), jnp.float32),
        pltpu.VMEM((RPT, D), jnp.float32),
        pltpu.VMEM((2, RPT, D), jnp.float32),
        pltpu.VMEM((2, RPT), jnp.float32),
        pltpu.VMEM((RPT, D), jnp.float32),
        pltpu.SemaphoreType.DMA,
        pltpu.SemaphoreType.DMA,
    ]

    def body(accin_hbm, cnt_hbm, h3_hbm, acc_out,
             table_sp, acc_sp, gidx_v, sidx_v, rows0_v, rows1_v,
             slab_v, a_v, d_v, t_v, sem0, sem1):
        cid = lax.axis_index("c")
        sid = lax.axis_index("s")
        wid = cid * NS + sid
        row0 = sid * RPT

        _zero_slab(slab_v, RPT)
        pltpu.sync_copy(slab_v, acc_sp.at[pl.ds(row0, RPT)])

        pltpu.sync_copy(accin_hbm.at[0, pl.ds(row0, RPT)], a_v.at[0])
        pltpu.sync_copy(accin_hbm.at[1, pl.ds(row0, RPT)], a_v.at[1])
        pltpu.sync_copy(cnt_hbm.at[0, pl.ds(row0, RPT)], d_v.at[0])
        pltpu.sync_copy(cnt_hbm.at[1, pl.ds(row0, RPT)], d_v.at[1])

        lanes = lax.iota(jnp.int32, D)

        def cgroup(g, _):
            c16 = d_v[0, pl.ds(g * D, D)] + d_v[1, pl.ds(g * D, D)]
            r16 = 1.0 / jnp.maximum(c16, 1.0)
            for j in range(D):
                i = g * D + j
                r = jnp.sum(jnp.where(lanes == j, r16, 0.0))
                t = (a_v[0, i, :] + a_v[1, i, :]) * r
                if leaky:
                    t = jnp.maximum(t, 0.01 * t)
                t_v[i, :] = t
            return 0
        lax.fori_loop(0, RPT // D, cgroup, 0)
        pltpu.sync_copy(t_v, table_sp.at[pl.ds(row0, RPT)])

        my_w = _stage_indices(h3_hbm, grow, srow, gidx_v, sidx_v, wid, wpw, tw)

        plsc.subcore_barrier()

        _window_pipeline(my_w, table_sp, acc_sp, gidx_v, sidx_v,
                         rows0_v, rows1_v, sem0, sem1)

        plsc.subcore_barrier()

        _copy_out(acc_sp, acc_out, cid, row0, slab_v)

    return pl.kernel(body, out_type=[_ACC], mesh=_MESH,
                     scratch_types=scratch, compiler_params=_PARAMS)


def _proj_body(x_ref, w_ref, b_ref, o_ref):
    o_ref[0:N, :] = (
        jnp.dot(x_ref[...], w_ref[...], preferred_element_type=jnp.float32)
        + b_ref[...]
    )
    o_ref[N:P, :] = jnp.zeros((P - N, D), jnp.float32)


def _final_body(acc_ref, dv_ref, w_ref, g_ref, o_ref):
    dv = dv_ref[0] + dv_ref[1]
    n2 = (acc_ref[0] + acc_ref[1]) * (1.0 / jnp.maximum(dv, 1.0))
    mask = (dv > 0.0).astype(jnp.float32)
    xa = jnp.concatenate([n2, mask], axis=1)
    logits = jnp.dot(xa, w_ref[...], preferred_element_type=jnp.float32)
    z = logits - jnp.max(logits, axis=1, keepdims=True)
    s = jnp.dot(jnp.exp(z), g_ref[...], preferred_element_type=jnp.float32)
    o_ref[...] = z - jnp.log(s)


def kernel(x, H, W1, b1, W2, b2):
    n_class = W2.shape[1]
    ni = H.shape[1]
    Hi = H.astype(jnp.int32)
    if ni % WIN:
        padn = WIN - ni % WIN
        Hi = jnp.concatenate(
            [Hi, jnp.full((2, padn), N, jnp.int32)], axis=1
        )
        ni += padn
    tw = ni // WIN
    wpw = _ceil_to(_ceil_to(ni, NW * WIN) // (NW * WIN), 8)
    hw = Hi.reshape(2 * tw, WIN)

    xp = pl.pallas_call(
        _proj_body, out_shape=jax.ShapeDtypeStruct((P, D), jnp.float32)
    )(x, W1, b1.reshape(1, D))

    sc_first = _make_sc_first(wpw, tw, grow=0, srow=tw)
    sc_mid = _make_sc_mid(wpw, tw, grow=tw, srow=0, leaky=False)
    sc_mid_leaky = _make_sc_mid(wpw, tw, grow=0, srow=tw, leaky=True)

    accA, deP, dvP = sc_first(xp, hw)
    (accB,) = sc_mid(accA, deP, hw)
    (accC,) = sc_mid_leaky(accB, dvP, hw)
    (accD,) = sc_mid(accC, deP, hw)

    p8 = P // 8
    eye8 = jnp.eye(8, dtype=jnp.float32)
    w_aug = jnp.concatenate(
        [
            jnp.kron(eye8, W2),
            jnp.kron(
                eye8,
                jnp.concatenate(
                    [b2.reshape(1, n_class),
                     jnp.zeros((D - 1, n_class), jnp.float32)]
                ),
            ),
        ],
        axis=0,
    )
    g_sum = jnp.kron(eye8, jnp.ones((n_class, n_class), jnp.float32))
    dv16 = jnp.repeat(dvP, D, axis=-1).reshape(NC, p8, 8 * D)
    out8 = pl.pallas_call(
        _final_body,
        out_shape=jax.ShapeDtypeStruct((p8, 8 * n_class), jnp.float32),
    )(accD.reshape(NC, p8, 8 * D), dv16, w_aug, g_sum)
    return out8.reshape(P, n_class)[:N]
